# baseline (speedup 1.0000x reference)
import jax
import jax.numpy as jnp
from jax.experimental import pallas as pl
from jax.experimental.pallas import tpu as pltpu

NUM_TOP = 8
NUM_CHUNKS = 8
CHUNK = 128
NEG_INF = float("-inf")


def _vq_kernel(x_ref, w_ref, b_ref, cb_ref,
               quant_ref, idx_ref, wts_ref, enc_ref, loss_ref):
    x_t = x_ref[:]
    logits = jnp.dot(x_t, w_ref[:], preferred_element_type=jnp.float32) + b_ref[:]
    tt, k = logits.shape
    iota128 = jax.lax.broadcasted_iota(jnp.int32, (tt, CHUNK), 1).astype(jnp.float32)
    iotas = [iota128 + float(c * CHUNK) for c in range(NUM_CHUNKS)]
    work = [logits[:, c * CHUNK:(c + 1) * CHUNK] for c in range(NUM_CHUNKS)]
    onehot = [jnp.zeros((tt, CHUNK), jnp.float32) for _ in range(NUM_CHUNKS)]
    esum = jnp.zeros((tt, 1), jnp.float32)
    idx_list = []
    e_list = []
    m0 = None
    for h in range(NUM_TOP):
        cm = work[0]
        for c in range(1, NUM_CHUNKS):
            cm = jnp.maximum(cm, work[c])
        m = jnp.max(cm, axis=-1, keepdims=True)
        masked = jnp.where(work[0] == m, iotas[0], float(k))
        for c in range(1, NUM_CHUNKS):
            masked = jnp.minimum(masked, jnp.where(work[c] == m, iotas[c], float(k)))
        idxf = jnp.min(masked, axis=-1, keepdims=True)
        if h == 0:
            m0 = m
        e = jnp.exp(m - m0)
        for c in range(NUM_CHUNKS):
            sel = iotas[c] == idxf
            onehot[c] = onehot[c] + jnp.where(sel, e, 0.0)
            work[c] = jnp.where(sel, NEG_INF, work[c])
        esum = esum + e
        idx_list.append(idxf)
        e_list.append(e)
    inv = 1.0 / esum
    enc = jnp.concatenate(onehot, axis=-1) * inv
    enc_ref[:] = enc
    idx_ref[:] = jnp.concatenate(idx_list, axis=-1).astype(jnp.int32)
    wts_ref[:] = jnp.concatenate(e_list, axis=-1) * inv
    quant = jnp.dot(enc, cb_ref[:], preferred_element_type=jnp.float32)
    quant_ref[:] = quant
    d = quant - x_t
    loss_ref[:] = jnp.sum(d * d).reshape(1, 1, 1)


@jax.jit
def kernel(x, W, b, codebook):
    B, T, D = x.shape
    K = codebook.shape[0]
    N = B * T
    tile = 256
    grid = N // tile
    xf = x.reshape(N, D)
    b2 = b.reshape(1, K)
    out_shapes = (
        jax.ShapeDtypeStruct((N, D), jnp.float32),
        jax.ShapeDtypeStruct((N, NUM_TOP), jnp.int32),
        jax.ShapeDtypeStruct((N, NUM_TOP), jnp.float32),
        jax.ShapeDtypeStruct((N, K), jnp.float32),
        jax.ShapeDtypeStruct((grid, 1, 1), jnp.float32),
    )
    quant, idx, wts, enc, lparts = pl.pallas_call(
        _vq_kernel,
        grid=(grid,),
        in_specs=[
            pl.BlockSpec((tile, D), lambda i: (i, 0)),
            pl.BlockSpec((D, K), lambda i: (0, 0)),
            pl.BlockSpec((1, K), lambda i: (0, 0)),
            pl.BlockSpec((K, D), lambda i: (0, 0)),
        ],
        out_specs=(
            pl.BlockSpec((tile, D), lambda i: (i, 0)),
            pl.BlockSpec((tile, NUM_TOP), lambda i: (i, 0)),
            pl.BlockSpec((tile, NUM_TOP), lambda i: (i, 0)),
            pl.BlockSpec((tile, K), lambda i: (i, 0)),
            pl.BlockSpec((1, 1, 1), lambda i: (i, 0, 0)),
        ),
        out_shape=out_shapes,
        compiler_params=pltpu.CompilerParams(
            dimension_semantics=("parallel",),
        ),
    )(xf, W, b2, codebook)
    loss = jnp.sum(lparts) * (1.25 / (N * D))
    return (
        quant.reshape(B, T, D),
        idx.reshape(B, T, NUM_TOP),
        wts.reshape(B, T, NUM_TOP),
        enc.reshape(B, T, K),
        loss,
    )


# speedup vs baseline: 1.0815x; 1.0815x over previous
import jax
import jax.numpy as jnp
from jax.experimental import pallas as pl
from jax.experimental.pallas import tpu as pltpu

NUM_TOP = 8
NUM_CHUNKS = 8
CHUNK = 128
NEG_INF = float("-inf")


def _vq_kernel(x_ref, w_ref, b_ref, cb_ref,
               quant_ref, idx_ref, wts_ref, enc_ref, loss_ref):
    x_t = x_ref[:]
    logits = jnp.dot(x_t, w_ref[:], preferred_element_type=jnp.float32) + b_ref[:]
    tt, k = logits.shape
    iota128 = jax.lax.broadcasted_iota(jnp.int32, (tt, CHUNK), 1).astype(jnp.float32)
    iotas = [iota128 + float(c * CHUNK) for c in range(NUM_CHUNKS)]
    work = [logits[:, c * CHUNK:(c + 1) * CHUNK] for c in range(NUM_CHUNKS)]
    onehot = [jnp.zeros((tt, CHUNK), jnp.float32) for _ in range(NUM_CHUNKS)]
    esum = jnp.zeros((tt, 1), jnp.float32)
    idx_list = []
    e_list = []
    m0 = None
    for h in range(NUM_TOP):
        cm = work[0]
        for c in range(1, NUM_CHUNKS):
            cm = jnp.maximum(cm, work[c])
        m = jnp.max(cm, axis=-1, keepdims=True)
        masked = jnp.where(work[0] == m, iotas[0], float(k))
        for c in range(1, NUM_CHUNKS):
            masked = jnp.minimum(masked, jnp.where(work[c] == m, iotas[c], float(k)))
        idxf = jnp.min(masked, axis=-1, keepdims=True)
        if h == 0:
            m0 = m
        e = jnp.exp(m - m0)
        for c in range(NUM_CHUNKS):
            sel = iotas[c] == idxf
            onehot[c] = onehot[c] + jnp.where(sel, e, 0.0)
            work[c] = jnp.where(sel, NEG_INF, work[c])
        esum = esum + e
        idx_list.append(idxf)
        e_list.append(e)
    inv = 1.0 / esum
    enc = jnp.concatenate(onehot, axis=-1) * inv
    enc_ref[:] = enc
    idx_ref[:] = jnp.concatenate(idx_list, axis=-1).astype(jnp.int32)
    wts_ref[:] = jnp.concatenate(e_list, axis=-1) * inv
    quant = jnp.dot(enc, cb_ref[:], preferred_element_type=jnp.float32)
    quant_ref[:] = quant
    d = quant - x_t
    loss_ref[:] = jnp.sum(d * d).reshape(1, 1, 1)


@jax.jit
def kernel(x, W, b, codebook):
    B, T, D = x.shape
    K = codebook.shape[0]
    N = B * T
    tile = 1024
    grid = N // tile
    xf = x.reshape(N, D)
    b2 = b.reshape(1, K)
    out_shapes = (
        jax.ShapeDtypeStruct((N, D), jnp.float32),
        jax.ShapeDtypeStruct((N, NUM_TOP), jnp.int32),
        jax.ShapeDtypeStruct((N, NUM_TOP), jnp.float32),
        jax.ShapeDtypeStruct((N, K), jnp.float32),
        jax.ShapeDtypeStruct((grid, 1, 1), jnp.float32),
    )
    quant, idx, wts, enc, lparts = pl.pallas_call(
        _vq_kernel,
        grid=(grid,),
        in_specs=[
            pl.BlockSpec((tile, D), lambda i: (i, 0)),
            pl.BlockSpec((D, K), lambda i: (0, 0)),
            pl.BlockSpec((1, K), lambda i: (0, 0)),
            pl.BlockSpec((K, D), lambda i: (0, 0)),
        ],
        out_specs=(
            pl.BlockSpec((tile, D), lambda i: (i, 0)),
            pl.BlockSpec((tile, NUM_TOP), lambda i: (i, 0)),
            pl.BlockSpec((tile, NUM_TOP), lambda i: (i, 0)),
            pl.BlockSpec((tile, K), lambda i: (i, 0)),
            pl.BlockSpec((1, 1, 1), lambda i: (i, 0, 0)),
        ),
        out_shape=out_shapes,
        compiler_params=pltpu.CompilerParams(
            dimension_semantics=("parallel",),
        ),
    )(xf, W, b2, codebook)
    loss = jnp.sum(lparts) * (1.25 / (N * D))
    return (
        quant.reshape(B, T, D),
        idx.reshape(B, T, NUM_TOP),
        wts.reshape(B, T, NUM_TOP),
        enc.reshape(B, T, K),
        loss,
    )
